# baseline (speedup 1.0000x reference)
import functools

import jax
import jax.numpy as jnp
from jax import lax
from jax.experimental import pallas as pl
from jax.experimental.pallas import tpu as pltpu
from jax.experimental.pallas import tpu_sc as plsc

D_MODEL = 64
SCALE = 8.0

NC = 2
NS = 16
NW = NC * NS
LANES = 16

C = 128
NCHUNK = 200
NBUF = 4


def _scale_rows(rows):
    def body(i, carry):
        for k in range(4):
            r = i * 4 + k
            for j in range(D_MODEL // LANES):
                sl = (r, pl.ds(j * LANES, LANES))
                rows[sl] = rows[sl] * SCALE
        return carry
    lax.fori_loop(0, C // 4, body, 0)


@functools.partial(
    pl.kernel,
    out_type=jax.ShapeDtypeStruct((NW, NCHUNK, C, D_MODEL), jnp.float32),
    mesh=plsc.VectorSubcoreMesh(core_axis_name="c", subcore_axis_name="s"),
    compiler_params=pltpu.CompilerParams(use_tc_tiling_on_sc=False),
    scratch_types=[
        pltpu.VMEM((NCHUNK, C), jnp.int32),
        pltpu.VMEM((NBUF, C, D_MODEL), jnp.float32),
        pltpu.SemaphoreType.DMA((NBUF,)),
        pltpu.SemaphoreType.DMA((NBUF,)),
    ],
)
def _emb_lookup(table_hbm, x_hbm, out_hbm, idx_v, rows_v, gsem, ssem):
    wid = lax.axis_index("s") * NC + lax.axis_index("c")

    pltpu.sync_copy(x_hbm.at[wid], idx_v)

    for b in range(NBUF):
        pltpu.async_copy(table_hbm.at[idx_v.at[b]], rows_v.at[b], gsem.at[b])

    def process(c, b):
        pltpu.make_async_copy(
            table_hbm.at[idx_v.at[c]], rows_v.at[b], gsem.at[b]).wait()
        _scale_rows(rows_v.at[b])
        pltpu.async_copy(rows_v.at[b], out_hbm.at[wid, c], ssem.at[b])

    def outer(c0, carry):
        for b in range(NBUF):
            c = c0 * NBUF + b
            process(c, b)
            pltpu.make_async_copy(
                rows_v.at[b], out_hbm.at[wid, c], ssem.at[b]).wait()
            pltpu.async_copy(
                table_hbm.at[idx_v.at[c + NBUF]], rows_v.at[b], gsem.at[b])
        return carry
    lax.fori_loop(0, NCHUNK // NBUF - 1, outer, 0)

    for b in range(NBUF):
        c = NCHUNK - NBUF + b
        process(c, b)
        pltpu.make_async_copy(
            rows_v.at[b], out_hbm.at[wid, c], ssem.at[b]).wait()


def kernel(x, table):
    xr = x.astype(jnp.int32).reshape(NW, NCHUNK, C)
    out = _emb_lookup(table, xr)
    return out.reshape(x.shape[0], x.shape[1], D_MODEL)
